# baseline (speedup 1.0000x reference)
import functools
import math

import jax
import jax.numpy as jnp
from jax.experimental import pallas as pl
from jax.experimental.pallas import tpu as pltpu

B = 16
C = 3
H = 224
P = 16
E = 768
NH = 12
NC = 1000
NEXP = 6
TOPK = 2
HID = 3072
NPATCH = (H // P) ** 2
T = B * NPATCH

_BT = 448
_SQRT2 = math.sqrt(2.0)

_BA = 128
_NB = (T * TOPK + NEXP * (_BA - 1) + _BA - 1) // _BA
_PADT = _NB * _BA
_TPAD = T + 8


def _moe_ffn_kernel(be_ref, x_ref, w1_ref, b1_ref, w2_ref, b2_ref,
                    g_ref, out_ref):
    x = x_ref[...]
    h = jax.lax.dot_general(x, w1_ref[0], (((1,), (1,)), ((), ())),
                            preferred_element_type=jnp.float32)
    h = h + b1_ref[0, 0]
    h = 0.5 * h * (1.0 + jax.lax.erf(h / _SQRT2))
    eo = jax.lax.dot_general(h, w2_ref[0], (((1,), (1,)), ((), ())),
                             preferred_element_type=jnp.float32)
    eo = eo + b2_ref[0, 0]
    out_ref[...] = eo * g_ref[...]


def _moe_sparse(flat, flat_i, flat_p, exp_w1, exp_b1, exp_w2, exp_b2):
    ids = flat_i.reshape(-1)
    oneh = (ids[:, None] == jnp.arange(NEXP, dtype=ids.dtype)[None, :])
    incl = jnp.cumsum(oneh.astype(jnp.int32), axis=0)
    rank = jnp.take_along_axis(incl, ids[:, None], axis=1)[:, 0] - 1
    counts = incl[-1]
    padded = ((counts + _BA - 1) // _BA) * _BA
    off = jnp.concatenate([jnp.zeros((1,), jnp.int32),
                           jnp.cumsum(padded).astype(jnp.int32)])
    pos = off[ids] + rank
    tok = (jnp.arange(T * TOPK, dtype=jnp.int32) // TOPK)
    row_token = jnp.full((_PADT,), T, jnp.int32).at[pos].set(tok)
    row_gate = jnp.zeros((_PADT,), jnp.float32).at[pos].set(flat_p.reshape(-1))
    starts = jnp.arange(_NB, dtype=jnp.int32) * _BA
    block_expert = jnp.minimum(
        jnp.searchsorted(off[1:], starts, side='right'),
        NEXP - 1).astype(jnp.int32)

    z_pad = jnp.concatenate([flat, jnp.zeros((_TPAD - T, E), flat.dtype)], axis=0)
    x_sorted = jnp.take(z_pad, row_token, axis=0)

    grid_spec = pltpu.PrefetchScalarGridSpec(
        num_scalar_prefetch=1,
        grid=(_NB,),
        in_specs=[
            pl.BlockSpec((_BA, E), lambda i, be: (i, 0)),
            pl.BlockSpec((1, HID, E), lambda i, be: (be[i], 0, 0)),
            pl.BlockSpec((1, 1, HID), lambda i, be: (be[i], 0, 0)),
            pl.BlockSpec((1, E, HID), lambda i, be: (be[i], 0, 0)),
            pl.BlockSpec((1, 1, E), lambda i, be: (be[i], 0, 0)),
            pl.BlockSpec((_BA, 1), lambda i, be: (i, 0)),
        ],
        out_specs=pl.BlockSpec((_BA, E), lambda i, be: (i, 0)),
    )
    buf = pl.pallas_call(
        _moe_ffn_kernel,
        grid_spec=grid_spec,
        out_shape=jax.ShapeDtypeStruct((_PADT, E), jnp.float32),
    )(block_expert, x_sorted, exp_w1,
      exp_b1.reshape(NEXP, 1, HID), exp_w2, exp_b2.reshape(NEXP, 1, E),
      row_gate[:, None])
    return jnp.take(buf, pos, axis=0).reshape(T, TOPK, E).sum(axis=1)


def _moe_dense_kernel(z_ref, w1_ref, b1_ref, w2_ref, b2_ref, gates_ref, out_ref):
    e = pl.program_id(1)
    z = z_ref[...]
    h = jax.lax.dot_general(z, w1_ref[0], (((1,), (1,)), ((), ())),
                            preferred_element_type=jnp.float32)
    h = h + b1_ref[0, 0]
    h = 0.5 * h * (1.0 + jax.lax.erf(h / _SQRT2))
    eo = jax.lax.dot_general(h, w2_ref[0], (((1,), (1,)), ((), ())),
                             preferred_element_type=jnp.float32)
    eo = eo + b2_ref[0, 0]
    lane = jax.lax.broadcasted_iota(jnp.int32, (_BT, NEXP), 1)
    g = jnp.sum(jnp.where(lane == e, gates_ref[...], 0.0), axis=1, keepdims=True)
    contrib = eo * g

    @pl.when(e == 0)
    def _init():
        out_ref[...] = contrib

    @pl.when(e != 0)
    def _acc():
        out_ref[...] += contrib


def _moe_dense(flat, gates, exp_w1, exp_b1, exp_w2, exp_b2):
    grid = (T // _BT, NEXP)
    return pl.pallas_call(
        _moe_dense_kernel,
        grid=grid,
        in_specs=[
            pl.BlockSpec((_BT, E), lambda t, e: (t, 0)),
            pl.BlockSpec((1, HID, E), lambda t, e: (e, 0, 0)),
            pl.BlockSpec((1, 1, HID), lambda t, e: (e, 0, 0)),
            pl.BlockSpec((1, E, HID), lambda t, e: (e, 0, 0)),
            pl.BlockSpec((1, 1, E), lambda t, e: (e, 0, 0)),
            pl.BlockSpec((_BT, NEXP), lambda t, e: (t, 0)),
        ],
        out_specs=pl.BlockSpec((_BT, E), lambda t, e: (t, 0)),
        out_shape=jax.ShapeDtypeStruct((T, E), jnp.float32),
    )(flat, exp_w1, exp_b1.reshape(NEXP, 1, HID), exp_w2,
      exp_b2.reshape(NEXP, 1, E), gates)


def _layernorm(x, g, b):
    m = jnp.mean(x, axis=-1, keepdims=True)
    v = jnp.mean((x - m) ** 2, axis=-1, keepdims=True)
    return (x - m) / jnp.sqrt(v + 1e-5) * g + b


def kernel(x, patch_w, patch_b, pos_embed, ln1_g, ln1_b, attn_in_w, attn_in_b,
           attn_out_w, attn_out_b, router_w, router_b, exp_w1, exp_b1, exp_w2,
           exp_b2, ln2_g, ln2_b, head_w, head_b):
    Bn = x.shape[0]
    hp = H // P
    patches = x.reshape(Bn, C, hp, P, hp, P).transpose(0, 2, 4, 1, 3, 5)
    patches = patches.reshape(Bn, hp * hp, C * P * P)
    z = patches @ patch_w.reshape(E, C * P * P).T + patch_b
    z = z + pos_embed

    zn = _layernorm(z, ln1_g, ln1_b)
    qkv = zn @ attn_in_w.T + attn_in_b
    q, k, v = jnp.split(qkv, 3, axis=-1)
    dh = E // NH

    def split_heads(t):
        return t.reshape(Bn, -1, NH, dh).transpose(0, 2, 1, 3)

    q = split_heads(q)
    k = split_heads(k)
    v = split_heads(v)
    att = jax.nn.softmax(
        jnp.einsum('bhqd,bhkd->bhqk', q, k) / jnp.sqrt(jnp.float32(dh)), axis=-1)
    ao = jnp.einsum('bhqk,bhkd->bhqd', att, v).transpose(0, 2, 1, 3).reshape(Bn, -1, E)
    ao = ao @ attn_out_w.T + attn_out_b
    z = z + ao

    logits = z @ router_w.T + router_b
    probs = jax.nn.softmax(logits, axis=-1)
    topk_p, topk_i = jax.lax.top_k(probs, TOPK)
    flat = z.reshape(T, E)
    flat_i = topk_i.reshape(T, TOPK)
    flat_p = topk_p.reshape(T, TOPK)

    out = _moe_sparse(flat, flat_i, flat_p, exp_w1, exp_b1, exp_w2, exp_b2)

    z = out.reshape(Bn, -1, E)
    z = _layernorm(z, ln2_g, ln2_b)
    pooled = jnp.mean(z, axis=1)
    return pooled @ head_w.T + head_b


# speedup vs baseline: 1.1131x; 1.1131x over previous
import functools
import math

import jax
import jax.numpy as jnp
from jax.experimental import pallas as pl
from jax.experimental.pallas import tpu as pltpu

B = 16
C = 3
H = 224
P = 16
E = 768
NH = 12
NC = 1000
NEXP = 6
TOPK = 2
HID = 3072
NPATCH = (H // P) ** 2
T = B * NPATCH

_BT = 448
_SQRT2 = math.sqrt(2.0)

_BA = 256
_NB = (T * TOPK + NEXP * (_BA - 1) + _BA - 1) // _BA
_PADT = _NB * _BA
_TPAD = T + 8


def _moe_ffn_kernel(be_ref, x_ref, w1_ref, b1_ref, w2_ref, b2_ref,
                    g_ref, out_ref):
    x = x_ref[...]
    h = jax.lax.dot_general(x, w1_ref[0], (((1,), (1,)), ((), ())),
                            preferred_element_type=jnp.float32)
    h = h + b1_ref[0, 0]
    h = 0.5 * h * (1.0 + jax.lax.erf(h / _SQRT2))
    eo = jax.lax.dot_general(h, w2_ref[0], (((1,), (1,)), ((), ())),
                             preferred_element_type=jnp.float32)
    eo = eo + b2_ref[0, 0]
    out_ref[...] = eo * g_ref[...]


def _moe_sparse(flat, flat_i, flat_p, exp_w1, exp_b1, exp_w2, exp_b2):
    ids = flat_i.reshape(-1)
    oneh = (ids[:, None] == jnp.arange(NEXP, dtype=ids.dtype)[None, :])
    incl = jnp.cumsum(oneh.astype(jnp.int32), axis=0)
    rank = jnp.take_along_axis(incl, ids[:, None], axis=1)[:, 0] - 1
    counts = incl[-1]
    padded = ((counts + _BA - 1) // _BA) * _BA
    off = jnp.concatenate([jnp.zeros((1,), jnp.int32),
                           jnp.cumsum(padded).astype(jnp.int32)])
    pos = off[ids] + rank
    tok = (jnp.arange(T * TOPK, dtype=jnp.int32) // TOPK)
    row_token = jnp.full((_PADT,), T, jnp.int32).at[pos].set(tok)
    row_gate = jnp.zeros((_PADT,), jnp.float32).at[pos].set(flat_p.reshape(-1))
    starts = jnp.arange(_NB, dtype=jnp.int32) * _BA
    block_expert = jnp.minimum(
        jnp.searchsorted(off[1:], starts, side='right'),
        NEXP - 1).astype(jnp.int32)

    z_pad = jnp.concatenate([flat, jnp.zeros((_TPAD - T, E), flat.dtype)], axis=0)
    x_sorted = jnp.take(z_pad, row_token, axis=0)

    grid_spec = pltpu.PrefetchScalarGridSpec(
        num_scalar_prefetch=1,
        grid=(_NB,),
        in_specs=[
            pl.BlockSpec((_BA, E), lambda i, be: (i, 0)),
            pl.BlockSpec((1, HID, E), lambda i, be: (be[i], 0, 0)),
            pl.BlockSpec((1, 1, HID), lambda i, be: (be[i], 0, 0)),
            pl.BlockSpec((1, E, HID), lambda i, be: (be[i], 0, 0)),
            pl.BlockSpec((1, 1, E), lambda i, be: (be[i], 0, 0)),
            pl.BlockSpec((_BA, 1), lambda i, be: (i, 0)),
        ],
        out_specs=pl.BlockSpec((_BA, E), lambda i, be: (i, 0)),
    )
    buf = pl.pallas_call(
        _moe_ffn_kernel,
        grid_spec=grid_spec,
        out_shape=jax.ShapeDtypeStruct((_PADT, E), jnp.float32),
    )(block_expert, x_sorted, exp_w1,
      exp_b1.reshape(NEXP, 1, HID), exp_w2, exp_b2.reshape(NEXP, 1, E),
      row_gate[:, None])
    return jnp.take(buf, pos, axis=0).reshape(T, TOPK, E).sum(axis=1)


def _moe_dense_kernel(z_ref, w1_ref, b1_ref, w2_ref, b2_ref, gates_ref, out_ref):
    e = pl.program_id(1)
    z = z_ref[...]
    h = jax.lax.dot_general(z, w1_ref[0], (((1,), (1,)), ((), ())),
                            preferred_element_type=jnp.float32)
    h = h + b1_ref[0, 0]
    h = 0.5 * h * (1.0 + jax.lax.erf(h / _SQRT2))
    eo = jax.lax.dot_general(h, w2_ref[0], (((1,), (1,)), ((), ())),
                             preferred_element_type=jnp.float32)
    eo = eo + b2_ref[0, 0]
    lane = jax.lax.broadcasted_iota(jnp.int32, (_BT, NEXP), 1)
    g = jnp.sum(jnp.where(lane == e, gates_ref[...], 0.0), axis=1, keepdims=True)
    contrib = eo * g

    @pl.when(e == 0)
    def _init():
        out_ref[...] = contrib

    @pl.when(e != 0)
    def _acc():
        out_ref[...] += contrib


def _moe_dense(flat, gates, exp_w1, exp_b1, exp_w2, exp_b2):
    grid = (T // _BT, NEXP)
    return pl.pallas_call(
        _moe_dense_kernel,
        grid=grid,
        in_specs=[
            pl.BlockSpec((_BT, E), lambda t, e: (t, 0)),
            pl.BlockSpec((1, HID, E), lambda t, e: (e, 0, 0)),
            pl.BlockSpec((1, 1, HID), lambda t, e: (e, 0, 0)),
            pl.BlockSpec((1, E, HID), lambda t, e: (e, 0, 0)),
            pl.BlockSpec((1, 1, E), lambda t, e: (e, 0, 0)),
            pl.BlockSpec((_BT, NEXP), lambda t, e: (t, 0)),
        ],
        out_specs=pl.BlockSpec((_BT, E), lambda t, e: (t, 0)),
        out_shape=jax.ShapeDtypeStruct((T, E), jnp.float32),
    )(flat, exp_w1, exp_b1.reshape(NEXP, 1, HID), exp_w2,
      exp_b2.reshape(NEXP, 1, E), gates)


def _layernorm(x, g, b):
    m = jnp.mean(x, axis=-1, keepdims=True)
    v = jnp.mean((x - m) ** 2, axis=-1, keepdims=True)
    return (x - m) / jnp.sqrt(v + 1e-5) * g + b


def kernel(x, patch_w, patch_b, pos_embed, ln1_g, ln1_b, attn_in_w, attn_in_b,
           attn_out_w, attn_out_b, router_w, router_b, exp_w1, exp_b1, exp_w2,
           exp_b2, ln2_g, ln2_b, head_w, head_b):
    Bn = x.shape[0]
    hp = H // P
    patches = x.reshape(Bn, C, hp, P, hp, P).transpose(0, 2, 4, 1, 3, 5)
    patches = patches.reshape(Bn, hp * hp, C * P * P)
    z = patches @ patch_w.reshape(E, C * P * P).T + patch_b
    z = z + pos_embed

    zn = _layernorm(z, ln1_g, ln1_b)
    qkv = zn @ attn_in_w.T + attn_in_b
    q, k, v = jnp.split(qkv, 3, axis=-1)
    dh = E // NH

    def split_heads(t):
        return t.reshape(Bn, -1, NH, dh).transpose(0, 2, 1, 3)

    q = split_heads(q)
    k = split_heads(k)
    v = split_heads(v)
    att = jax.nn.softmax(
        jnp.einsum('bhqd,bhkd->bhqk', q, k) / jnp.sqrt(jnp.float32(dh)), axis=-1)
    ao = jnp.einsum('bhqk,bhkd->bhqd', att, v).transpose(0, 2, 1, 3).reshape(Bn, -1, E)
    ao = ao @ attn_out_w.T + attn_out_b
    z = z + ao

    logits = z @ router_w.T + router_b
    probs = jax.nn.softmax(logits, axis=-1)
    topk_p, topk_i = jax.lax.top_k(probs, TOPK)
    flat = z.reshape(T, E)
    flat_i = topk_i.reshape(T, TOPK)
    flat_p = topk_p.reshape(T, TOPK)

    out = _moe_sparse(flat, flat_i, flat_p, exp_w1, exp_b1, exp_w2, exp_b2)

    z = out.reshape(Bn, -1, E)
    z = _layernorm(z, ln2_g, ln2_b)
    pooled = jnp.mean(z, axis=1)
    return pooled @ head_w.T + head_b


# speedup vs baseline: 1.3290x; 1.1939x over previous
import functools
import math

import jax
import jax.numpy as jnp
from jax.experimental import pallas as pl
from jax.experimental.pallas import tpu as pltpu

B = 16
C = 3
H = 224
P = 16
E = 768
NH = 12
NC = 1000
NEXP = 6
TOPK = 2
HID = 3072
NPATCH = (H // P) ** 2
T = B * NPATCH

_BT = 448
_SQRT2 = math.sqrt(2.0)

_BA = 256
_NB = (T * TOPK + NEXP * (_BA - 1) + _BA - 1) // _BA
_PADT = _NB * _BA
_TPAD = T + 8


def _moe_ffn_kernel(be_ref, x_ref, w1_ref, b1_ref, w2_ref, b2_ref,
                    g_ref, out_ref):
    x = x_ref[...]
    h = jax.lax.dot_general(x, w1_ref[0], (((1,), (1,)), ((), ())),
                            preferred_element_type=jnp.float32)
    h = h + b1_ref[0, 0]
    h = 0.5 * h * (1.0 + jax.lax.erf(h / _SQRT2))
    eo = jax.lax.dot_general(h, w2_ref[0], (((1,), (1,)), ((), ())),
                             preferred_element_type=jnp.float32)
    eo = eo + b2_ref[0, 0]
    out_ref[...] = eo * g_ref[...]


def _moe_sparse(flat, flat_i, flat_p, exp_w1, exp_b1, exp_w2, exp_b2):
    ids = flat_i.reshape(-1)
    oneh = (ids[:, None] == jnp.arange(NEXP, dtype=ids.dtype)[None, :])
    incl = jnp.cumsum(oneh.astype(jnp.int32), axis=0)
    rank = jnp.take_along_axis(incl, ids[:, None], axis=1)[:, 0] - 1
    counts = incl[-1]
    padded = ((counts + _BA - 1) // _BA) * _BA
    off = jnp.concatenate([jnp.zeros((1,), jnp.int32),
                           jnp.cumsum(padded).astype(jnp.int32)])
    pos = off[ids] + rank
    tok = (jnp.arange(T * TOPK, dtype=jnp.int32) // TOPK)
    row_token = jnp.full((_PADT,), T, jnp.int32).at[pos].set(tok)
    row_gate = jnp.zeros((_PADT,), jnp.float32).at[pos].set(flat_p.reshape(-1))
    starts = jnp.arange(_NB, dtype=jnp.int32) * _BA
    block_expert = jnp.minimum(
        jnp.searchsorted(off[1:], starts, side='right'),
        NEXP - 1).astype(jnp.int32)

    z_pad = jnp.concatenate([flat, jnp.zeros((_TPAD - T, E), flat.dtype)], axis=0)
    x_sorted = jnp.take(z_pad, row_token, axis=0)

    grid_spec = pltpu.PrefetchScalarGridSpec(
        num_scalar_prefetch=1,
        grid=(_NB,),
        in_specs=[
            pl.BlockSpec((_BA, E), lambda i, be: (i, 0)),
            pl.BlockSpec((1, HID, E), lambda i, be: (be[i], 0, 0)),
            pl.BlockSpec((1, 1, HID), lambda i, be: (be[i], 0, 0)),
            pl.BlockSpec((1, E, HID), lambda i, be: (be[i], 0, 0)),
            pl.BlockSpec((1, 1, E), lambda i, be: (be[i], 0, 0)),
            pl.BlockSpec((_BA, 1), lambda i, be: (i, 0)),
        ],
        out_specs=pl.BlockSpec((_BA, E), lambda i, be: (i, 0)),
    )
    buf = pl.pallas_call(
        _moe_ffn_kernel,
        grid_spec=grid_spec,
        out_shape=jax.ShapeDtypeStruct((_PADT, E), jnp.float32),
    )(block_expert, x_sorted, exp_w1,
      exp_b1.reshape(NEXP, 1, HID), exp_w2, exp_b2.reshape(NEXP, 1, E),
      row_gate[:, None])
    pos2 = pos.reshape(T, TOPK)
    return jnp.take(buf, pos2[:, 0], axis=0) + jnp.take(buf, pos2[:, 1], axis=0)


def _moe_dense_kernel(z_ref, w1_ref, b1_ref, w2_ref, b2_ref, gates_ref, out_ref):
    e = pl.program_id(1)
    z = z_ref[...]
    h = jax.lax.dot_general(z, w1_ref[0], (((1,), (1,)), ((), ())),
                            preferred_element_type=jnp.float32)
    h = h + b1_ref[0, 0]
    h = 0.5 * h * (1.0 + jax.lax.erf(h / _SQRT2))
    eo = jax.lax.dot_general(h, w2_ref[0], (((1,), (1,)), ((), ())),
                             preferred_element_type=jnp.float32)
    eo = eo + b2_ref[0, 0]
    lane = jax.lax.broadcasted_iota(jnp.int32, (_BT, NEXP), 1)
    g = jnp.sum(jnp.where(lane == e, gates_ref[...], 0.0), axis=1, keepdims=True)
    contrib = eo * g

    @pl.when(e == 0)
    def _init():
        out_ref[...] = contrib

    @pl.when(e != 0)
    def _acc():
        out_ref[...] += contrib


def _moe_dense(flat, gates, exp_w1, exp_b1, exp_w2, exp_b2):
    grid = (T // _BT, NEXP)
    return pl.pallas_call(
        _moe_dense_kernel,
        grid=grid,
        in_specs=[
            pl.BlockSpec((_BT, E), lambda t, e: (t, 0)),
            pl.BlockSpec((1, HID, E), lambda t, e: (e, 0, 0)),
            pl.BlockSpec((1, 1, HID), lambda t, e: (e, 0, 0)),
            pl.BlockSpec((1, E, HID), lambda t, e: (e, 0, 0)),
            pl.BlockSpec((1, 1, E), lambda t, e: (e, 0, 0)),
            pl.BlockSpec((_BT, NEXP), lambda t, e: (t, 0)),
        ],
        out_specs=pl.BlockSpec((_BT, E), lambda t, e: (t, 0)),
        out_shape=jax.ShapeDtypeStruct((T, E), jnp.float32),
    )(flat, exp_w1, exp_b1.reshape(NEXP, 1, HID), exp_w2,
      exp_b2.reshape(NEXP, 1, E), gates)


def _layernorm(x, g, b):
    m = jnp.mean(x, axis=-1, keepdims=True)
    v = jnp.mean((x - m) ** 2, axis=-1, keepdims=True)
    return (x - m) / jnp.sqrt(v + 1e-5) * g + b


def kernel(x, patch_w, patch_b, pos_embed, ln1_g, ln1_b, attn_in_w, attn_in_b,
           attn_out_w, attn_out_b, router_w, router_b, exp_w1, exp_b1, exp_w2,
           exp_b2, ln2_g, ln2_b, head_w, head_b):
    Bn = x.shape[0]
    hp = H // P
    patches = x.reshape(Bn, C, hp, P, hp, P).transpose(0, 2, 4, 1, 3, 5)
    patches = patches.reshape(Bn, hp * hp, C * P * P)
    z = patches @ patch_w.reshape(E, C * P * P).T + patch_b
    z = z + pos_embed

    zn = _layernorm(z, ln1_g, ln1_b)
    qkv = zn @ attn_in_w.T + attn_in_b
    q, k, v = jnp.split(qkv, 3, axis=-1)
    dh = E // NH

    def split_heads(t):
        return t.reshape(Bn, -1, NH, dh).transpose(0, 2, 1, 3)

    q = split_heads(q)
    k = split_heads(k)
    v = split_heads(v)
    att = jax.nn.softmax(
        jnp.einsum('bhqd,bhkd->bhqk', q, k) / jnp.sqrt(jnp.float32(dh)), axis=-1)
    ao = jnp.einsum('bhqk,bhkd->bhqd', att, v).transpose(0, 2, 1, 3).reshape(Bn, -1, E)
    ao = ao @ attn_out_w.T + attn_out_b
    z = z + ao

    logits = z @ router_w.T + router_b
    probs = jax.nn.softmax(logits, axis=-1)
    topk_p, topk_i = jax.lax.top_k(probs, TOPK)
    flat = z.reshape(T, E)
    flat_i = topk_i.reshape(T, TOPK)
    flat_p = topk_p.reshape(T, TOPK)

    out = _moe_sparse(flat, flat_i, flat_p, exp_w1, exp_b1, exp_w2, exp_b2)

    z = out.reshape(Bn, -1, E)
    z = _layernorm(z, ln2_g, ln2_b)
    pooled = jnp.mean(z, axis=1)
    return pooled @ head_w.T + head_b
